# baseline (speedup 1.0000x reference)
import functools

import jax
import jax.numpy as jnp
from jax import lax
from jax.experimental import pallas as pl
from jax.experimental.pallas import tpu as pltpu
from jax.experimental.pallas import tpu_sc as plsc

NC = 2
NS = 16
L = 16
NW = NC * NS
D = 64
CH = 64
W = 2 * D


def _body(pk_hbm, ere_hbm, eim_hbm, rel_hbm, out_hbm,
          idx_v0, idx_v1, ehre_v0, ehre_v1, ehim_v0, ehim_v1,
          etre_v0, etre_v1, etim_v0, etim_v1, rl_v0, rl_v1,
          out_v, sem0, sem1, *, n_chunks):
    wid = lax.axis_index("s") * NC + lax.axis_index("c")
    rows0 = jnp.arange(L, dtype=jnp.int32)
    idx_bufs = (idx_v0, idx_v1)
    ehre_bufs = (ehre_v0, ehre_v1)
    ehim_bufs = (ehim_v0, ehim_v1)
    etre_bufs = (etre_v0, etre_v1)
    etim_bufs = (etim_v0, etim_v1)
    rl_bufs = (rl_v0, rl_v1)
    sems = (sem0, sem1)

    def fire(c, b):
        pltpu.sync_copy(pk_hbm.at[wid * n_chunks + c], idx_bufs[b])
        pltpu.async_copy(ere_hbm.at[idx_bufs[b].at[0]], ehre_bufs[b], sems[b])
        pltpu.async_copy(eim_hbm.at[idx_bufs[b].at[0]], ehim_bufs[b], sems[b])
        pltpu.async_copy(ere_hbm.at[idx_bufs[b].at[1]], etre_bufs[b], sems[b])
        pltpu.async_copy(eim_hbm.at[idx_bufs[b].at[1]], etim_bufs[b], sems[b])
        pltpu.async_copy(rel_hbm.at[idx_bufs[b].at[2]], rl_bufs[b], sems[b])

    def drain(b):
        pltpu.make_async_copy(ere_hbm.at[pl.ds(0, CH)], ehre_bufs[b], sems[b]).wait()
        pltpu.make_async_copy(eim_hbm.at[pl.ds(0, CH)], ehim_bufs[b], sems[b]).wait()
        pltpu.make_async_copy(ere_hbm.at[pl.ds(0, CH)], etre_bufs[b], sems[b]).wait()
        pltpu.make_async_copy(eim_hbm.at[pl.ds(0, CH)], etim_bufs[b], sems[b]).wait()
        pltpu.make_async_copy(rel_hbm.at[pl.ds(0, CH)], rl_bufs[b], sems[b]).wait()

    fire(0, 0)
    for c in range(n_chunks):
        b = c % 2
        if c + 1 < n_chunks:
            fire(c + 1, 1 - b)
        drain(b)
        idx_v = idx_bufs[b]
        ehre_v, ehim_v = ehre_bufs[b], ehim_bufs[b]
        etre_v, etim_v = etre_bufs[b], etim_bufs[b]
        rl_v = rl_bufs[b]

        def group_body(g, _):
            rows = g * L + rows0
            hoff = idx_v[3, pl.ds(g * L, L)] * D
            toff = idx_v[4, pl.ds(g * L, L)] * D

            def d_body(d, acc):
                cols = jnp.full((L,), d, dtype=jnp.int32)
                hcols = hoff + cols
                tcols = toff + cols
                ehre = plsc.load_gather(ehre_v, [rows, hcols])
                ehim = plsc.load_gather(ehim_v, [rows, hcols])
                etre = plsc.load_gather(etre_v, [rows, tcols])
                etim = plsc.load_gather(etim_v, [rows, tcols])
                rre = plsc.load_gather(rl_v, [rows, cols])
                rim = plsc.load_gather(rl_v, [rows, cols + D])
                return (acc + rre * (ehre * etre + ehim * etim)
                        + rim * (ehre * etim - ehim * etre))

            acc = lax.fori_loop(0, D, d_body, jnp.zeros((L,), jnp.float32))
            out_v[pl.ds(g * L, L)] = acc
            return 0

        lax.fori_loop(0, CH // L, group_body, 0)
        pltpu.sync_copy(out_v, out_hbm.at[pl.ds((wid * n_chunks + c) * CH, CH)])


def kernel(hs, rs, ts, ent_re, ent_im, rel_re, rel_im):
    batch = hs.shape[0]
    n_chunks = batch // NW // CH
    num_ent = ent_re.shape[0]
    ere2 = ent_re.reshape(num_ent // 2, W)
    eim2 = ent_im.reshape(num_ent // 2, W)
    rel = jnp.concatenate([rel_re, rel_im], axis=1)
    hp = lax.shift_right_logical(hs, 1)
    tp = lax.shift_right_logical(ts, 1)
    hsel = lax.bitwise_and(hs, 1)
    tsel = lax.bitwise_and(ts, 1)
    pk = jnp.stack([hp, tp, rs, hsel, tsel], axis=0)
    pk = pk.reshape(5, batch // CH, CH).transpose(1, 0, 2)
    mesh = plsc.VectorSubcoreMesh(core_axis_name="c", subcore_axis_name="s")
    k = pl.kernel(
        functools.partial(_body, n_chunks=n_chunks),
        out_type=jax.ShapeDtypeStruct((batch,), jnp.float32),
        mesh=mesh,
        compiler_params=pltpu.CompilerParams(needs_layout_passes=False),
        scratch_types=[
            pltpu.VMEM((5, CH), jnp.int32),
            pltpu.VMEM((5, CH), jnp.int32),
            pltpu.VMEM((CH, W), jnp.float32),
            pltpu.VMEM((CH, W), jnp.float32),
            pltpu.VMEM((CH, W), jnp.float32),
            pltpu.VMEM((CH, W), jnp.float32),
            pltpu.VMEM((CH, W), jnp.float32),
            pltpu.VMEM((CH, W), jnp.float32),
            pltpu.VMEM((CH, W), jnp.float32),
            pltpu.VMEM((CH, W), jnp.float32),
            pltpu.VMEM((CH, W), jnp.float32),
            pltpu.VMEM((CH, W), jnp.float32),
            pltpu.VMEM((CH,), jnp.float32),
            pltpu.SemaphoreType.DMA,
            pltpu.SemaphoreType.DMA,
        ],
    )
    return k(pk, ere2, eim2, rel)
